# baseline (speedup 1.0000x reference)
import functools
import math

import jax
import jax.numpy as jnp
from jax import lax
from jax.experimental import pallas as pl
from jax.experimental.pallas import tpu as pltpu
from jax.experimental.pallas import tpu_sc as plsc

EMB = 64
SCALE = math.sqrt(EMB)
LANES = 16
BLK = 128
NBUF = 4


@functools.lru_cache(maxsize=None)
def _make_kernel(S, B, NC, NS):
    NW = NC * NS
    n_blocks_total = (S * B) // BLK
    n_blocks = n_blocks_total // NW
    blocks_per_s = B // BLK
    mesh = plsc.VectorSubcoreMesh(core_axis_name="c", subcore_axis_name="s")

    @functools.partial(
        pl.kernel,
        mesh=mesh,
        compiler_params=pltpu.CompilerParams(
            use_tc_tiling_on_sc=True, needs_layout_passes=False
        ),
        out_type=jax.ShapeDtypeStruct((S, EMB, B), jnp.float32),
        scratch_types=[
            pltpu.VMEM((NBUF, BLK), jnp.int32),
            pltpu.VMEM((NBUF, BLK), jnp.int32),
            pltpu.VMEM((NBUF, BLK), jnp.int32),
            pltpu.VMEM((NBUF, BLK, 2 * EMB), jnp.float32),
            pltpu.VMEM((NBUF, EMB, BLK), jnp.float32),
            pltpu.SemaphoreType.DMA((NBUF,)),
            pltpu.SemaphoreType.DMA((NBUF,)),
            pltpu.SemaphoreType.DMA((NBUF,)),
        ],
    )
    def k(tokens_hbm, table_hbm, out_hbm, tbuf, qbuf, hbuf, gbuf, obuf,
          tsem, gsem, osem):
        wid = lax.axis_index("s") * NC + lax.axis_index("c")
        base = wid * n_blocks

        def tok_slice(j):
            bid = base + j
            s = bid // blocks_per_s
            c = bid % blocks_per_s
            return tokens_hbm.at[s, pl.ds(c * BLK, BLK)]

        def start_tok(j, b):
            pltpu.async_copy(tok_slice(j), tbuf.at[b], tsem.at[b])

        def wait_tok(j, b):
            pltpu.make_async_copy(tok_slice(j), tbuf.at[b], tsem.at[b]).wait()

        def split(b):
            for kk in range(BLK // LANES):
                sl = pl.ds(kk * LANES, LANES)
                tv = tbuf[b, sl]
                hbuf[b, sl] = lax.shift_left(jnp.bitwise_and(tv, 1), 6)
                qbuf[b, sl] = lax.shift_right_logical(tv, 1)

        def start_gather(b):
            pltpu.async_copy(table_hbm.at[qbuf.at[b]], gbuf.at[b], gsem.at[b])

        def wait_gather(b):
            pltpu.make_async_copy(
                table_hbm.at[qbuf.at[b]], gbuf.at[b], gsem.at[b]
            ).wait()

        def out_slice(j):
            bid = base + j
            s = bid // blocks_per_s
            c = bid % blocks_per_s
            return out_hbm.at[s, :, pl.ds(c * BLK, BLK)]

        def start_write(j, b):
            pltpu.async_copy(obuf.at[b], out_slice(j), osem.at[b])

        def wait_write(j, b):
            pltpu.make_async_copy(obuf.at[b], out_slice(j), osem.at[b]).wait()

        rows = [
            lax.iota(jnp.int32, LANES) + kk * LANES for kk in range(BLK // LANES)
        ]

        start_tok(0, 0)
        start_tok(1, 1)
        wait_tok(0, 0)
        split(0)
        start_gather(0)
        start_tok(2, 2)
        wait_tok(1, 1)
        split(1)
        start_gather(1)

        def block_body(j2, carry):
            for b in range(NBUF):
                jj = j2 * NBUF + b
                b2 = (b + 2) % NBUF
                b3 = (b + 3) % NBUF

                @pl.when(jj + 2 < n_blocks)
                def _():
                    wait_tok(jj + 2, b2)
                    split(b2)
                    start_gather(b2)

                @pl.when(jj + 3 < n_blocks)
                def _():
                    start_tok(jj + 3, b3)

                wait_gather(b)

                @pl.when(jj >= NBUF)
                def _():
                    wait_write(jj - NBUF, b)

                halves = [
                    hbuf[b, pl.ds(kk * LANES, LANES)]
                    for kk in range(BLK // LANES)
                ]

                def d_body(d4, c2):
                    for dd in range(8):
                        d = d4 * 8 + dd
                        for kk in range(BLK // LANES):
                            feat = jnp.bitwise_and(rows[kk] + d, EMB - 1)
                            gcols = halves[kk] + feat
                            vals = plsc.load_gather(
                                gbuf.at[b], [rows[kk], gcols]
                            )
                            plsc.store_scatter(
                                obuf.at[b], [feat, rows[kk]], vals * SCALE
                            )
                    return c2

                lax.fori_loop(0, EMB // 8, d_body, 0)
                start_write(jj, b)
            return carry

        lax.fori_loop(0, n_blocks // NBUF, block_body, 0)
        for b in range(NBUF):
            wait_write(n_blocks - NBUF + b, b)

    return k


@functools.lru_cache(maxsize=None)
def _make_prep(V, NC, NS):
    NW = NC * NS
    full_slabs = V // BLK
    n_iter = -(-full_slabs // NW)
    n_iter = ((n_iter + NBUF - 1) // NBUF) * NBUF
    mesh = plsc.VectorSubcoreMesh(core_axis_name="c", subcore_axis_name="s")

    @functools.partial(
        pl.kernel,
        mesh=mesh,
        compiler_params=pltpu.CompilerParams(
            use_tc_tiling_on_sc=True, needs_layout_passes=False
        ),
        out_type=jax.ShapeDtypeStruct((V // 2, 2 * EMB), jnp.float32),
        scratch_types=[
            pltpu.VMEM((NBUF, EMB, BLK), jnp.float32),
            pltpu.VMEM((NBUF, EMB, BLK), jnp.float32),
            pltpu.SemaphoreType.DMA((NBUF,)),
            pltpu.SemaphoreType.DMA((NBUF,)),
        ],
    )
    def pk(tab_hbm, tail_hbm, out_hbm, gbuf, obuf, rsem, wsem):
        wid = lax.axis_index("s") * NC + lax.axis_index("c")
        tail_rows = (V - full_slabs * BLK) // 2

        @pl.when(wid == 0)
        def _():
            pltpu.sync_copy(
                tail_hbm, out_hbm.at[pl.ds(full_slabs * (BLK // 2), tail_rows)]
            )

        def slab_of(m):
            return wid + m * NW

        def start_read(m, b):
            slab = slab_of(m)

            @pl.when(slab < full_slabs)
            def _():
                pltpu.async_copy(
                    tab_hbm.at[:, pl.ds(slab * BLK, BLK)], gbuf.at[b], rsem.at[b]
                )

        def wait_read(m, b):
            slab = slab_of(m)

            @pl.when(slab < full_slabs)
            def _():
                pltpu.make_async_copy(
                    tab_hbm.at[:, pl.ds(slab * BLK, BLK)], gbuf.at[b], rsem.at[b]
                ).wait()

        def start_write(m, b):
            slab = slab_of(m)

            @pl.when(slab < full_slabs)
            def _():
                pltpu.async_copy(
                    obuf.at[b], out_hbm.at[pl.ds(slab * EMB, EMB)], wsem.at[b]
                )

        def wait_write(m, b):
            slab = slab_of(m)

            @pl.when(slab < full_slabs)
            def _():
                pltpu.make_async_copy(
                    obuf.at[b], out_hbm.at[pl.ds(slab * EMB, EMB)], wsem.at[b]
                ).wait()

        dvec = [
            jnp.mod(lax.iota(jnp.int32, LANES) + kk * LANES, EMB)
            for kk in range(BLK // LANES)
        ]
        hib = [
            lax.div(lax.iota(jnp.int32, LANES) + kk * LANES, EMB)
            for kk in range(BLK // LANES)
        ]
        li = lax.iota(jnp.int32, LANES)
        lconst = [
            lax.iota(jnp.int32, LANES) + kk * LANES for kk in range(BLK // LANES)
        ]

        start_read(0, 0)

        def iter_body(m2, carry):
            for b in range(NBUF):
                m = m2 * NBUF + b
                nb = (b + 1) % NBUF
                slab = slab_of(m)

                @pl.when(m + 1 < n_iter)
                def _():
                    start_read(m + 1, nb)

                wait_read(m, b)

                @pl.when(m >= NBUF)
                def _():
                    wait_write(m - NBUF, b)

                def qq_body(q4, c2):
                    for qi in range(8):
                        qq = q4 * 8 + qi
                        for kk in range(BLK // LANES):
                            qrow = jnp.bitwise_and(li + qq, EMB - 1)
                            cols = hib[kk] + 2 * qrow
                            vals = plsc.load_gather(
                                gbuf.at[b], [dvec[kk], cols]
                            )
                            plsc.store_scatter(
                                obuf.at[b], [qrow, lconst[kk]], vals
                            )
                    return c2

                @pl.when(slab < full_slabs)
                def _():
                    lax.fori_loop(0, EMB // 8, qq_body, 0)

                start_write(m, b)
            return carry

        lax.fori_loop(0, n_iter // NBUF, iter_body, 0)
        for b in range(NBUF):
            wait_write(n_iter - NBUF + b, b)

    return pk


def kernel(tokens, table):
    B, S = tokens.shape
    V = table.shape[0]
    info = plsc.get_sparse_core_info()
    prep = _make_prep(V, info.num_cores, info.num_subcores)
    k = _make_kernel(S, B, info.num_cores, info.num_subcores)
    tokens_t = tokens.T.astype(jnp.int32)
    full = (V // BLK) * BLK
    tail2 = table[full:].reshape((V - full) // 2, 2 * table.shape[1])
    table2 = prep(jnp.transpose(table), tail2)
    out = k(tokens_t, table2)
    return jnp.transpose(out, (2, 0, 1))


# speedup vs baseline: 1.0134x; 1.0134x over previous
import functools
import math

import jax
import jax.numpy as jnp
from jax import lax
from jax.experimental import pallas as pl
from jax.experimental.pallas import tpu as pltpu
from jax.experimental.pallas import tpu_sc as plsc

EMB = 64
SCALE = math.sqrt(EMB)
LANES = 16
BLK = 128
NBUF = 2


@functools.lru_cache(maxsize=None)
def _make_kernel(S, B, NC, NS):
    NW = NC * NS
    n_blocks_total = (S * B) // BLK
    n_blocks = n_blocks_total // NW
    blocks_per_s = B // BLK
    mesh = plsc.VectorSubcoreMesh(core_axis_name="c", subcore_axis_name="s")

    @functools.partial(
        pl.kernel,
        mesh=mesh,
        compiler_params=pltpu.CompilerParams(
            use_tc_tiling_on_sc=True, needs_layout_passes=False
        ),
        out_type=jax.ShapeDtypeStruct((S, EMB, B), jnp.float32),
        scratch_types=[
            pltpu.VMEM((NBUF, BLK), jnp.int32),
            pltpu.VMEM((NBUF, BLK), jnp.int32),
            pltpu.VMEM((NBUF, BLK), jnp.int32),
            pltpu.VMEM((NBUF, BLK, 2 * EMB), jnp.float32),
            pltpu.VMEM((NBUF, EMB, BLK), jnp.float32),
            pltpu.SemaphoreType.DMA((NBUF,)),
            pltpu.SemaphoreType.DMA((NBUF,)),
            pltpu.SemaphoreType.DMA((NBUF,)),
        ],
    )
    def k(tokens_hbm, table_hbm, out_hbm, tbuf, qbuf, hbuf, gbuf, obuf,
          tsem, gsem, osem):
        wid = lax.axis_index("s") * NC + lax.axis_index("c")
        base = wid * n_blocks

        def tok_slice(j):
            bid = base + j
            s = bid // blocks_per_s
            c = bid % blocks_per_s
            return tokens_hbm.at[s, pl.ds(c * BLK, BLK)]

        def start_tok(j, b):
            pltpu.async_copy(tok_slice(j), tbuf.at[b], tsem.at[b])

        def wait_tok(j, b):
            pltpu.make_async_copy(tok_slice(j), tbuf.at[b], tsem.at[b]).wait()

        def split(b):
            for kk in range(BLK // LANES):
                sl = pl.ds(kk * LANES, LANES)
                tv = tbuf[b, sl]
                hbuf[b, sl] = lax.shift_left(jnp.bitwise_and(tv, 1), 6)
                qbuf[b, sl] = lax.shift_right_logical(tv, 1)

        def start_gather(b):
            pltpu.async_copy(table_hbm.at[qbuf.at[b]], gbuf.at[b], gsem.at[b])

        def wait_gather(b):
            pltpu.make_async_copy(
                table_hbm.at[qbuf.at[b]], gbuf.at[b], gsem.at[b]
            ).wait()

        def out_slice(j):
            bid = base + j
            s = bid // blocks_per_s
            c = bid % blocks_per_s
            return out_hbm.at[s, :, pl.ds(c * BLK, BLK)]

        def start_write(j, b):
            pltpu.async_copy(obuf.at[b], out_slice(j), osem.at[b])

        def wait_write(j, b):
            pltpu.make_async_copy(obuf.at[b], out_slice(j), osem.at[b]).wait()

        rows = [
            lax.iota(jnp.int32, LANES) + kk * LANES for kk in range(BLK // LANES)
        ]

        start_tok(0, 0)
        wait_tok(0, 0)
        split(0)
        start_gather(0)
        start_tok(1, 1)

        def block_body(j2, carry):
            for b in range(NBUF):
                jj = j2 * NBUF + b
                nb = (b + 1) % NBUF

                @pl.when(jj + 1 < n_blocks)
                def _():
                    wait_tok(jj + 1, nb)
                    split(nb)
                    start_gather(nb)

                @pl.when(jj + 2 < n_blocks)
                def _():
                    start_tok(jj + 2, b)

                wait_gather(b)

                @pl.when(jj >= NBUF)
                def _():
                    wait_write(jj - NBUF, b)

                halves = [
                    hbuf[b, pl.ds(kk * LANES, LANES)]
                    for kk in range(BLK // LANES)
                ]

                def d_body(d4, c2):
                    for dd in range(4):
                        d = d4 * 4 + dd
                        for kk in range(BLK // LANES):
                            feat = jnp.bitwise_and(rows[kk] + d, EMB - 1)
                            gcols = halves[kk] + feat
                            vals = plsc.load_gather(
                                gbuf.at[b], [rows[kk], gcols]
                            )
                            plsc.store_scatter(
                                obuf.at[b], [feat, rows[kk]], vals * SCALE
                            )
                    return c2

                lax.fori_loop(0, EMB // 4, d_body, 0)
                start_write(jj, b)
            return carry

        lax.fori_loop(0, n_blocks // NBUF, block_body, 0)
        for b in range(NBUF):
            wait_write(n_blocks - NBUF + b, b)

    return k


@functools.lru_cache(maxsize=None)
def _make_prep(V, NC, NS):
    NW = NC * NS
    full_slabs = V // BLK
    n_iter = -(-full_slabs // NW)
    n_iter = ((n_iter + NBUF - 1) // NBUF) * NBUF
    mesh = plsc.VectorSubcoreMesh(core_axis_name="c", subcore_axis_name="s")

    @functools.partial(
        pl.kernel,
        mesh=mesh,
        compiler_params=pltpu.CompilerParams(
            use_tc_tiling_on_sc=True, needs_layout_passes=False
        ),
        out_type=jax.ShapeDtypeStruct((V // 2, 2 * EMB), jnp.float32),
        scratch_types=[
            pltpu.VMEM((NBUF, EMB, BLK), jnp.float32),
            pltpu.VMEM((NBUF, EMB, BLK), jnp.float32),
            pltpu.SemaphoreType.DMA((NBUF,)),
            pltpu.SemaphoreType.DMA((NBUF,)),
        ],
    )
    def pk(tab_hbm, tail_hbm, out_hbm, gbuf, obuf, rsem, wsem):
        wid = lax.axis_index("s") * NC + lax.axis_index("c")
        tail_rows = (V - full_slabs * BLK) // 2

        @pl.when(wid == 0)
        def _():
            pltpu.sync_copy(
                tail_hbm, out_hbm.at[pl.ds(full_slabs * (BLK // 2), tail_rows)]
            )

        def slab_of(m):
            return wid + m * NW

        def start_read(m, b):
            slab = slab_of(m)

            @pl.when(slab < full_slabs)
            def _():
                pltpu.async_copy(
                    tab_hbm.at[:, pl.ds(slab * BLK, BLK)], gbuf.at[b], rsem.at[b]
                )

        def wait_read(m, b):
            slab = slab_of(m)

            @pl.when(slab < full_slabs)
            def _():
                pltpu.make_async_copy(
                    tab_hbm.at[:, pl.ds(slab * BLK, BLK)], gbuf.at[b], rsem.at[b]
                ).wait()

        def start_write(m, b):
            slab = slab_of(m)

            @pl.when(slab < full_slabs)
            def _():
                pltpu.async_copy(
                    obuf.at[b], out_hbm.at[pl.ds(slab * EMB, EMB)], wsem.at[b]
                )

        def wait_write(m, b):
            slab = slab_of(m)

            @pl.when(slab < full_slabs)
            def _():
                pltpu.make_async_copy(
                    obuf.at[b], out_hbm.at[pl.ds(slab * EMB, EMB)], wsem.at[b]
                ).wait()

        dvec = [
            jnp.mod(lax.iota(jnp.int32, LANES) + kk * LANES, EMB)
            for kk in range(BLK // LANES)
        ]
        hib = [
            lax.div(lax.iota(jnp.int32, LANES) + kk * LANES, EMB)
            for kk in range(BLK // LANES)
        ]
        li = lax.iota(jnp.int32, LANES)
        lconst = [
            lax.iota(jnp.int32, LANES) + kk * LANES for kk in range(BLK // LANES)
        ]

        start_read(0, 0)

        def iter_body(m2, carry):
            for b in range(NBUF):
                m = m2 * NBUF + b
                nb = (b + 1) % NBUF
                slab = slab_of(m)

                @pl.when(m + 1 < n_iter)
                def _():
                    start_read(m + 1, nb)

                wait_read(m, b)

                @pl.when(m >= NBUF)
                def _():
                    wait_write(m - NBUF, b)

                def qq_body(q4, c2):
                    for qi in range(4):
                        qq = q4 * 4 + qi
                        for kk in range(BLK // LANES):
                            qrow = jnp.bitwise_and(li + qq, EMB - 1)
                            cols = hib[kk] + 2 * qrow
                            vals = plsc.load_gather(
                                gbuf.at[b], [dvec[kk], cols]
                            )
                            plsc.store_scatter(
                                obuf.at[b], [qrow, lconst[kk]], vals
                            )
                    return c2

                @pl.when(slab < full_slabs)
                def _():
                    lax.fori_loop(0, EMB // 4, qq_body, 0)

                start_write(m, b)
            return carry

        lax.fori_loop(0, n_iter // NBUF, iter_body, 0)
        for b in range(NBUF):
            wait_write(n_iter - NBUF + b, b)

    return pk


def kernel(tokens, table):
    B, S = tokens.shape
    V = table.shape[0]
    info = plsc.get_sparse_core_info()
    prep = _make_prep(V, info.num_cores, info.num_subcores)
    k = _make_kernel(S, B, info.num_cores, info.num_subcores)
    tokens_t = tokens.T.astype(jnp.int32)
    full = (V // BLK) * BLK
    tail2 = table[full:].reshape((V - full) // 2, 2 * table.shape[1])
    table2 = prep(jnp.transpose(table), tail2)
    out = k(tokens_t, table2)
    return jnp.transpose(out, (2, 0, 1))
